# Initial kernel scaffold; baseline (speedup 1.0000x reference)
import jax
import jax.numpy as jnp
from jax.experimental import pallas as pl


def kernel(x, edge_index, ricci_curvature, params):
    raise NotImplementedError("write your pallas kernel here")


# speedup vs baseline: 1.5641x; 1.5641x over previous
import functools

import jax
import jax.numpy as jnp
from jax import lax
from jax.experimental import pallas as pl
from jax.experimental.pallas import tpu as pltpu
from jax.experimental.pallas import tpu_sc as plsc

N = 50000
E = 800000
H = 64
HP = 128
NP = N // 2

NC = 2
NS = 16
NW = NC * NS

GW = 200
GPW = E // NW


@functools.lru_cache(maxsize=None)
def _gather_kernel():
    mesh = plsc.VectorSubcoreMesh(core_axis_name="c", subcore_axis_name="s")

    @functools.partial(
        pl.kernel,
        mesh=mesh,
        out_type=(
            jax.ShapeDtypeStruct((E, HP), jnp.float32),
            jax.ShapeDtypeStruct((E, HP), jnp.float32),
        ),
        scratch_types=[
            pltpu.VMEM((GW,), jnp.int32),
            pltpu.VMEM((GW, HP), jnp.float32),
            pltpu.SemaphoreType.DMA,
        ],
    )
    def k(table, src_hbm, dst_hbm, out_src, out_dst, idx_v, rows_v, sem):
        wid = lax.axis_index("s") * NC + lax.axis_index("c")
        base = wid * GPW

        def body(w, carry):
            off = base + w * GW
            pltpu.sync_copy(src_hbm.at[pl.ds(off, GW)], idx_v)
            pltpu.async_copy(table.at[idx_v], rows_v, sem).wait()
            pltpu.sync_copy(rows_v, out_src.at[pl.ds(off, GW)])
            pltpu.sync_copy(dst_hbm.at[pl.ds(off, GW)], idx_v)
            pltpu.async_copy(table.at[idx_v], rows_v, sem).wait()
            pltpu.sync_copy(rows_v, out_dst.at[pl.ds(off, GW)])
            return carry

        lax.fori_loop(0, GPW // GW, body, 0)

    return k


def _sc_gather(table, srch, dsth):
    return _gather_kernel()(table, srch, dsth)


Q_ROWS = 6272
Q3_ROWS = NP - 3 * Q_ROWS
DUMMY_ROWS = 512
SP_ROWS = Q_ROWS + DUMMY_ROWS
SW = 400
EPS_SC = E // NS


@functools.lru_cache(maxsize=None)
def _scatter_kernel():
    mesh = plsc.VectorSubcoreMesh(core_axis_name="c", subcore_axis_name="s")

    @functools.partial(
        pl.kernel,
        mesh=mesh,
        out_type=jax.ShapeDtypeStruct((NP, HP), jnp.float32),
        scratch_types=[
            pltpu.VMEM((SW,), jnp.int32),
            pltpu.VMEM((SW,), jnp.int32),
            pltpu.VMEM((SW, HP), jnp.float32),
            pltpu.VMEM_SHARED((SP_ROWS, HP), jnp.float32),
        ],
    )
    def k(em_hbm, dsth_hbm, out_hbm, dstv, locv, emv, acc):
        c = lax.axis_index("c")
        s = lax.axis_index("s")
        lanes = lax.iota(jnp.int32, 16)
        zv = jnp.zeros((16,), jnp.float32)

        for p in range(2):
            q = p * 2 + c
            lo = q * Q_ROWS
            hi = jnp.where(q == 3, Q3_ROWS, Q_ROWS)

            def zrow(i, carry):
                for j in range(HP // 16):
                    emv[i, pl.ds(j * 16, 16)] = zv
                return carry

            lax.fori_loop(0, SW, zrow, 0)
            zrows = SP_ROWS // NS
            zbase = s * zrows
            pltpu.sync_copy(emv, acc.at[pl.ds(zbase, SW)])
            pltpu.sync_copy(emv.at[pl.ds(0, zrows - SW)],
                            acc.at[pl.ds(zbase + SW, zrows - SW)])
            plsc.subcore_barrier()

            def body(w, carry):
                off = s * EPS_SC + w * SW
                pltpu.sync_copy(dsth_hbm.at[pl.ds(off, SW)], dstv)
                pltpu.sync_copy(em_hbm.at[pl.ds(off, SW)], emv)

                def ib(i, carry2):
                    d = dstv[pl.ds(i * 16, 16)]
                    local = d - lo
                    inb = (local >= 0) & (local < hi)
                    grp = (w * (SW // 16) + i) & (DUMMY_ROWS // 16 - 1)
                    dummy = Q_ROWS + grp * 16 + lanes
                    locv[pl.ds(i * 16, 16)] = jnp.where(inb, local, dummy)
                    return carry2

                lax.fori_loop(0, SW // 16, ib, 0)
                pltpu.sync_copy(emv, acc.at[locv], add=True)
                return carry

            lax.fori_loop(0, EPS_SC // SW, body, 0)
            plsc.subcore_barrier()

            @pl.when((q < 3) | (s < 15))
            def _():
                pltpu.sync_copy(acc.at[pl.ds(s * 392, 392)],
                                out_hbm.at[pl.ds(lo + s * 392, 392)])

            @pl.when((q == 3) & (s == 15))
            def _():
                pltpu.sync_copy(acc.at[pl.ds(15 * 392, 304)],
                                out_hbm.at[pl.ds(lo + 15 * 392, 304)])

            plsc.subcore_barrier()

    return k


def _sc_scatter(em, dsth):
    return _scatter_kernel()(em, dsth)


BNP = 1000
BE = 2000


def _full_spec(shape):
    nd = len(shape)
    return pl.BlockSpec(shape, lambda i: (0,) * nd)


def _proj_body(x_ref, w_ref, b_ref, o_ref):
    o_ref[...] = jnp.dot(x_ref[...], w_ref[...],
                         preferred_element_type=jnp.float32) + b_ref[...]


def _proj(x2, w2, b2):
    return pl.pallas_call(
        _proj_body,
        grid=(NP // BNP,),
        in_specs=[
            pl.BlockSpec((BNP, 256), lambda i: (i, 0)),
            _full_spec((256, HP)),
            _full_spec((1, HP)),
        ],
        out_specs=pl.BlockSpec((BNP, HP), lambda i: (i, 0)),
        out_shape=jax.ShapeDtypeStruct((NP, HP), jnp.float32),
    )(x2, w2, b2)


def _sel(g, p):
    return g[:, 0:H] * (1.0 - p) + g[:, H:2 * H] * p


def _pack_em(em, p):
    return jnp.concatenate([em * (1.0 - p), em * p], axis=1)


def _edge_first_body(gd, gs, r, pd, ps, wgx, bg, rvg, wm1, bm, rvm, w2, b2,
                     em):
    pdv = pd[...]
    xiv = _sel(gd[...], pdv)
    xjv = _sel(gs[...], ps[...])
    X = jnp.concatenate([xiv, xjv], axis=1)
    rr = r[...]
    hg = jnp.tanh(jnp.dot(X, wgx[...], preferred_element_type=jnp.float32)
                  + bg[...] + rr * rvg[...])
    hm = jax.nn.relu(jnp.dot(xjv, wm1[...], preferred_element_type=jnp.float32)
                     + bm[...] + rr * rvm[...])
    S = jnp.dot(jnp.concatenate([hg, hm], axis=1), w2[...],
                preferred_element_type=jnp.float32) + b2[...]
    em[...] = _pack_em(jax.nn.sigmoid(S[:, 0:1]) * S[:, 1:1 + H], pdv)


def _edge_first(gd, gs, r, pd, ps, wgx, bg, rvg, wm1, bm, rvm, w2, b2):
    return pl.pallas_call(
        _edge_first_body,
        grid=(E // BE,),
        in_specs=[
            pl.BlockSpec((BE, HP), lambda i: (i, 0)),
            pl.BlockSpec((BE, HP), lambda i: (i, 0)),
            pl.BlockSpec((BE, 1), lambda i: (i, 0)),
            pl.BlockSpec((BE, 1), lambda i: (i, 0)),
            pl.BlockSpec((BE, 1), lambda i: (i, 0)),
            _full_spec((128, H)), _full_spec((1, H)), _full_spec((1, H)),
            _full_spec((H, H)), _full_spec((1, H)), _full_spec((1, H)),
            _full_spec((128, 1 + H)), _full_spec((1, 1 + H)),
        ],
        out_specs=pl.BlockSpec((BE, HP), lambda i: (i, 0)),
        out_shape=jax.ShapeDtypeStruct((E, HP), jnp.float32),
    )(gd, gs, r, pd, ps, wgx, bg, rvg, wm1, bm, rvm, w2, b2)


def _edge_mid_body(gd, gs, r, pd, ps, wgc, bgc, wc2, bc2, rvg, wm1, bm, rvm,
                   w2, b2, em, rn_ref):
    pdv = pd[...]
    xiv = _sel(gd[...], pdv)
    xjv = _sel(gs[...], ps[...])
    X = jnp.concatenate([xiv, xjv], axis=1)
    P = jnp.dot(X, wgc[...], preferred_element_type=jnp.float32) + bgc[...]
    hc = jnp.tanh(P[:, 0:H])
    t = jnp.tanh(jnp.dot(hc, wc2[...], preferred_element_type=jnp.float32)
                 + bc2[...])
    rn = 0.7 * r[...] + 0.3 * t
    hg = jnp.tanh(P[:, H:2 * H] + rn * rvg[...])
    hm = jax.nn.relu(jnp.dot(xjv, wm1[...], preferred_element_type=jnp.float32)
                     + bm[...] + rn * rvm[...])
    S = jnp.dot(jnp.concatenate([hg, hm], axis=1), w2[...],
                preferred_element_type=jnp.float32) + b2[...]
    em[...] = _pack_em(jax.nn.sigmoid(S[:, 0:1]) * S[:, 1:1 + H], pdv)
    rn_ref[...] = rn


def _edge_mid(gd, gs, r, pd, ps, wgc, bgc, wc2, bc2, rvg, wm1, bm, rvm,
              w2, b2):
    return pl.pallas_call(
        _edge_mid_body,
        grid=(E // BE,),
        in_specs=[
            pl.BlockSpec((BE, HP), lambda i: (i, 0)),
            pl.BlockSpec((BE, HP), lambda i: (i, 0)),
            pl.BlockSpec((BE, 1), lambda i: (i, 0)),
            pl.BlockSpec((BE, 1), lambda i: (i, 0)),
            pl.BlockSpec((BE, 1), lambda i: (i, 0)),
            _full_spec((128, 128)), _full_spec((1, 128)),
            _full_spec((H, 1)), _full_spec((1, 1)),
            _full_spec((1, H)),
            _full_spec((H, H)), _full_spec((1, H)), _full_spec((1, H)),
            _full_spec((128, 1 + H)), _full_spec((1, 1 + H)),
        ],
        out_specs=[
            pl.BlockSpec((BE, HP), lambda i: (i, 0)),
            pl.BlockSpec((BE, 1), lambda i: (i, 0)),
        ],
        out_shape=[
            jax.ShapeDtypeStruct((E, HP), jnp.float32),
            jax.ShapeDtypeStruct((E, 1), jnp.float32),
        ],
    )(gd, gs, r, pd, ps, wgc, bgc, wc2, bc2, rvg, wm1, bm, rvm, w2, b2)


def _edge_final_body(gd, gs, r, pd, ps, wce, bce, wc2, bc2, rve, we2, be2,
                     rn_ref, ew):
    xiv = _sel(gd[...], pd[...])
    xjv = _sel(gs[...], ps[...])
    X = jnp.concatenate([xiv, xjv], axis=1)
    P = jnp.dot(X, wce[...], preferred_element_type=jnp.float32) + bce[...]
    hc = jnp.tanh(P[:, 0:H])
    t = jnp.tanh(jnp.dot(hc, wc2[...], preferred_element_type=jnp.float32)
                 + bc2[...])
    rn = 0.7 * r[...] + 0.3 * t
    hw = jax.nn.relu(P[:, H:2 * H] + rn * rve[...])
    ew[...] = jax.nn.sigmoid(jnp.dot(hw, we2[...],
                                     preferred_element_type=jnp.float32)
                             + be2[...])
    rn_ref[...] = rn


def _edge_final(gd, gs, r, pd, ps, wce, bce, wc2, bc2, rve, we2, be2):
    return pl.pallas_call(
        _edge_final_body,
        grid=(E // BE,),
        in_specs=[
            pl.BlockSpec((BE, HP), lambda i: (i, 0)),
            pl.BlockSpec((BE, HP), lambda i: (i, 0)),
            pl.BlockSpec((BE, 1), lambda i: (i, 0)),
            pl.BlockSpec((BE, 1), lambda i: (i, 0)),
            pl.BlockSpec((BE, 1), lambda i: (i, 0)),
            _full_spec((128, 128)), _full_spec((1, 128)),
            _full_spec((H, 1)), _full_spec((1, 1)),
            _full_spec((1, H)),
            _full_spec((H, 1)), _full_spec((1, 1)),
        ],
        out_specs=[
            pl.BlockSpec((BE, 1), lambda i: (i, 0)),
            pl.BlockSpec((BE, 1), lambda i: (i, 0)),
        ],
        out_shape=[
            jax.ShapeDtypeStruct((E, 1), jnp.float32),
            jax.ShapeDtypeStruct((E, 1), jnp.float32),
        ],
    )(gd, gs, r, pd, ps, wce, bce, wc2, bc2, rve, we2, be2)


def _ln_half(y, sc, bi):
    mu = jnp.mean(y, axis=1, keepdims=True)
    var = jnp.mean((y - mu) ** 2, axis=1, keepdims=True)
    return (y - mu) / jnp.sqrt(var + 1e-5) * sc + bi


def _update_body(h, aggr, wut, wub, bu2, sc, bi, o_ref):
    hh = h[...]
    u = jax.nn.relu(jnp.dot(hh, wut[...], preferred_element_type=jnp.float32)
                    + jnp.dot(aggr[...], wub[...],
                              preferred_element_type=jnp.float32)
                    + bu2[...])
    y = hh + u
    scv = sc[...]
    biv = bi[...]
    o_ref[...] = jnp.concatenate(
        [_ln_half(y[:, 0:H], scv, biv), _ln_half(y[:, H:2 * H], scv, biv)],
        axis=1)


def _update(h, aggr, wut, wub, bu2, sc, bi):
    return pl.pallas_call(
        _update_body,
        grid=(NP // BNP,),
        in_specs=[
            pl.BlockSpec((BNP, HP), lambda i: (i, 0)),
            pl.BlockSpec((BNP, HP), lambda i: (i, 0)),
            _full_spec((HP, HP)), _full_spec((HP, HP)), _full_spec((1, HP)),
            _full_spec((1, H)), _full_spec((1, H)),
        ],
        out_specs=pl.BlockSpec((BNP, HP), lambda i: (i, 0)),
        out_shape=jax.ShapeDtypeStruct((NP, HP), jnp.float32),
    )(h, aggr, wut, wub, bu2, sc, bi)


def _final_body(h, wn1, bn1, wn2, bn2, wg1, bg1, wg2, bg2,
                risk, hsum, gsum):
    pid = pl.program_id(0)
    hh = h[...]
    q = jax.nn.relu(jnp.dot(hh, wn1[...], preferred_element_type=jnp.float32)
                    + bn1[...])
    risk[...] = jax.nn.sigmoid(jnp.dot(q, wn2[...],
                                       preferred_element_type=jnp.float32)
                               + bn2[...])
    blksum = jnp.sum(hh[:, 0:H] + hh[:, H:2 * H], axis=0, keepdims=True)

    @pl.when(pid == 0)
    def _():
        hsum[...] = blksum

    @pl.when(pid > 0)
    def _():
        hsum[...] = hsum[...] + blksum

    @pl.when(pid == NP // BNP - 1)
    def _():
        m = hsum[...] * (1.0 / N)
        g = jax.nn.relu(jnp.dot(m, wg1[...], preferred_element_type=jnp.float32)
                        + bg1[...])
        gsum[...] = jnp.dot(g, wg2[...], preferred_element_type=jnp.float32) \
            + bg2[...]


def _node_final(h, wn1, bn1, wn2, bn2, wg1, bg1, wg2, bg2):
    return pl.pallas_call(
        _final_body,
        grid=(NP // BNP,),
        in_specs=[
            pl.BlockSpec((BNP, HP), lambda i: (i, 0)),
            _full_spec((HP, H)), _full_spec((1, H)),
            _full_spec((H, 2)), _full_spec((1, 2)),
            _full_spec((H, 32)), _full_spec((1, 32)),
            _full_spec((32, 3)), _full_spec((1, 3)),
        ],
        out_specs=[
            pl.BlockSpec((BNP, 2), lambda i: (i, 0)),
            _full_spec((1, H)),
            _full_spec((1, 3)),
        ],
        out_shape=[
            jax.ShapeDtypeStruct((NP, 2), jnp.float32),
            jax.ShapeDtypeStruct((1, H), jnp.float32),
            jax.ShapeDtypeStruct((1, 3), jnp.float32),
        ],
    )(h, wn1, bn1, wn2, bn2, wg1, bg1, wg2, bg2)


def _row(v):
    return v.reshape(1, -1)


def _bd(w):
    a, b = w.shape
    z = jnp.zeros((a, b), jnp.float32)
    return jnp.concatenate([
        jnp.concatenate([w, z], axis=1),
        jnp.concatenate([z, w], axis=1),
    ], axis=0)


def _pair(v):
    return jnp.concatenate([v, v]).reshape(1, -1)


def _pack_gm(lp):
    Wg, bg = lp['gate1']
    wg2, bg2 = lp['gate2']
    Wm, bm = lp['msg1']
    Wm2, bm2 = lp['msg2']
    wgx = Wg[0:2 * H]
    rvg = _row(Wg[2 * H])
    wm1 = Wm[0:H]
    rvm = _row(Wm[H])
    w2 = jnp.concatenate([
        jnp.concatenate([wg2, jnp.zeros((H, H), jnp.float32)], axis=1),
        jnp.concatenate([jnp.zeros((H, 1), jnp.float32), Wm2], axis=1),
    ], axis=0)
    b2 = jnp.concatenate([bg2, bm2]).reshape(1, 1 + H)
    return wgx, _row(bg), rvg, wm1, _row(bm), rvm, w2, b2


def _pack_curv(lp):
    Wc, bc = lp['curv1']
    wc2, bc2 = lp['curv2']
    wcx = jnp.concatenate([Wc[H:2 * H], Wc[0:H]], axis=0)
    return wcx, bc, wc2, bc2.reshape(1, 1)


def kernel(x, edge_index, ricci_curvature, params):
    src = edge_index[0]
    dst = edge_index[1]
    srch = lax.shift_right_logical(src, 1)
    dsth = lax.shift_right_logical(dst, 1)
    psrc = (src & 1).astype(jnp.float32).reshape(E, 1)
    pdst = (dst & 1).astype(jnp.float32).reshape(E, 1)
    r0 = ricci_curvature.reshape(E, 1)
    L = params['layers']

    win, bin_ = params['input_proj']
    x2 = x.reshape(NP, 256)
    h0 = _proj(x2, _bd(win), _pair(bin_))

    gs0, gd0 = _sc_gather(h0, srch, dsth)
    wgx, bg, rvg, wm1, bm, rvm, w2, b2 = _pack_gm(L[0])
    em1 = _edge_first(gd0, gs0, r0, pdst, psrc,
                      wgx, bg, rvg, wm1, bm, rvm, w2, b2)
    a1 = _sc_scatter(em1, dsth)

    def upd(h, a, lp):
        wu, bu = lp['upd']
        return _update(h, a, _bd(wu[0:H]), _bd(wu[H:2 * H]), _pair(bu),
                       _row(lp['ln_scale']), _row(lp['ln_bias']))

    h = upd(h0, a1, L[0])
    r = r0
    for li in (1, 2):
        gs, gd = _sc_gather(h, srch, dsth)
        wcx, bc, wc2, bc2 = _pack_curv(L[li - 1])
        wgx, bg, rvg, wm1, bm, rvm, w2, b2 = _pack_gm(L[li])
        wgc = jnp.concatenate([wcx, wgx], axis=1)
        bgc = jnp.concatenate([bc, bg.reshape(-1)]).reshape(1, 2 * H)
        em, r = _edge_mid(gd, gs, r, pdst, psrc, wgc, bgc, wc2, bc2, rvg,
                          wm1, bm, rvm, w2, b2)
        a = _sc_scatter(em, dsth)
        h = upd(h, a, L[li])

    gs, gd = _sc_gather(h, srch, dsth)
    wcx, bc, wc2, bc2 = _pack_curv(L[2])
    We, be = params['ew1']
    wex = jnp.concatenate([We[H:2 * H], We[0:H]], axis=0)
    rve = _row(We[2 * H])
    we2, be2 = params['ew2']
    wce = jnp.concatenate([wcx, wex], axis=1)
    bce = jnp.concatenate([bc, be]).reshape(1, 2 * H)
    r3, ewp = _edge_final(gd, gs, r, pdst, psrc, wce, bce, wc2, bc2, rve,
                          we2, be2.reshape(1, 1))

    wn1, bn1 = params['nr1']
    wn2, bn2 = params['nr2']
    wg1, bg1 = params['gs1']
    wg2, bg2 = params['gs2']
    wn2p = jnp.concatenate([
        jnp.concatenate([wn2, jnp.zeros((32, 1), jnp.float32)], axis=1),
        jnp.concatenate([jnp.zeros((32, 1), jnp.float32), wn2], axis=1),
    ], axis=0)
    risk, _, gsum = _node_final(
        h, _bd(wn1), _pair(bn1), wn2p,
        jnp.concatenate([bn2, bn2]).reshape(1, 2),
        wg1, _row(bg1), wg2, _row(bg2))

    return (h.reshape(N, H), ewp.reshape(E), risk.reshape(N),
            gsum.reshape(3), r3.reshape(E))


# speedup vs baseline: 1.6695x; 1.0674x over previous
import functools

import jax
import jax.numpy as jnp
from jax import lax
from jax.experimental import pallas as pl
from jax.experimental.pallas import tpu as pltpu
from jax.experimental.pallas import tpu_sc as plsc

N = 50000
E = 800000
H = 64
HP = 128
NP = N // 2

NC = 2
NS = 16
NW = NC * NS

GW = 200
GPW = E // NW


@functools.lru_cache(maxsize=None)
def _gather_kernel():
    mesh = plsc.VectorSubcoreMesh(core_axis_name="c", subcore_axis_name="s")

    @functools.partial(
        pl.kernel,
        mesh=mesh,
        out_type=(
            jax.ShapeDtypeStruct((E, HP), jnp.float32),
            jax.ShapeDtypeStruct((E, HP), jnp.float32),
        ),
        scratch_types=[
            pltpu.VMEM((GPW,), jnp.int32),
            pltpu.VMEM((GW, HP), jnp.float32),
            pltpu.VMEM((GW, HP), jnp.float32),
            pltpu.SemaphoreType.DMA,
            pltpu.SemaphoreType.DMA,
        ],
    )
    def k(table, src_hbm, dst_hbm, out_src, out_dst, idx_v, ra, rb, sa, sb):
        wid = lax.axis_index("s") * NC + lax.axis_index("c")
        base = wid * GPW
        nwin = GPW // GW

        def gat(w, buf, sem):
            return pltpu.async_copy(
                table.at[idx_v.at[pl.ds(w * GW, GW)]], buf, sem)

        def wait(w, buf, sem):
            pltpu.make_async_copy(
                table.at[idx_v.at[pl.ds(w * GW, GW)]], buf, sem).wait()

        for idx_hbm, out in ((src_hbm, out_src), (dst_hbm, out_dst)):
            pltpu.sync_copy(idx_hbm.at[pl.ds(base, GPW)], idx_v)
            gat(0, ra, sa)

            def body(i, carry):
                w0 = 2 * i
                gat(w0 + 1, rb, sb)
                wait(w0, ra, sa)
                pltpu.sync_copy(ra, out.at[pl.ds(base + w0 * GW, GW)])
                gat(w0 + 2, ra, sa)
                wait(w0 + 1, rb, sb)
                pltpu.sync_copy(rb, out.at[pl.ds(base + (w0 + 1) * GW, GW)])
                return carry

            lax.fori_loop(0, (nwin - 1) // 2, body, 0)
            wait(nwin - 1, ra, sa)
            pltpu.sync_copy(ra, out.at[pl.ds(base + (nwin - 1) * GW, GW)])

    return k


def _sc_gather(table, srch, dsth):
    return _gather_kernel()(table, srch, dsth)


Q_ROWS = 6272
Q3_ROWS = NP - 3 * Q_ROWS
DUMMY_ROWS = 512
SP_ROWS = Q_ROWS + DUMMY_ROWS
SW = 400
EPS_SC = E // NS


@functools.lru_cache(maxsize=None)
def _scatter_kernel():
    mesh = plsc.VectorSubcoreMesh(core_axis_name="c", subcore_axis_name="s")

    @functools.partial(
        pl.kernel,
        mesh=mesh,
        out_type=jax.ShapeDtypeStruct((NP, HP), jnp.float32),
        scratch_types=[
            pltpu.VMEM((SW,), jnp.int32),
            pltpu.VMEM((SW,), jnp.int32),
            pltpu.VMEM((SW, HP), jnp.float32),
            pltpu.VMEM_SHARED((SP_ROWS, HP), jnp.float32),
        ],
    )
    def k(em_hbm, dsth_hbm, out_hbm, dstv, locv, emv, acc):
        c = lax.axis_index("c")
        s = lax.axis_index("s")
        lanes = lax.iota(jnp.int32, 16)
        zv = jnp.zeros((16,), jnp.float32)

        for p in range(2):
            q = p * 2 + c
            lo = q * Q_ROWS
            hi = jnp.where(q == 3, Q3_ROWS, Q_ROWS)

            def zrow(i, carry):
                for j in range(HP // 16):
                    emv[i, pl.ds(j * 16, 16)] = zv
                return carry

            lax.fori_loop(0, SW, zrow, 0)
            zrows = SP_ROWS // NS
            zbase = s * zrows
            pltpu.sync_copy(emv, acc.at[pl.ds(zbase, SW)])
            pltpu.sync_copy(emv.at[pl.ds(0, zrows - SW)],
                            acc.at[pl.ds(zbase + SW, zrows - SW)])
            plsc.subcore_barrier()

            def body(w, carry):
                off = s * EPS_SC + w * SW
                pltpu.sync_copy(dsth_hbm.at[pl.ds(off, SW)], dstv)
                pltpu.sync_copy(em_hbm.at[pl.ds(off, SW)], emv)

                def ib(i, carry2):
                    d = dstv[pl.ds(i * 16, 16)]
                    local = d - lo
                    inb = (local >= 0) & (local < hi)
                    grp = (w * (SW // 16) + i) & (DUMMY_ROWS // 16 - 1)
                    dummy = Q_ROWS + grp * 16 + lanes
                    locv[pl.ds(i * 16, 16)] = jnp.where(inb, local, dummy)
                    return carry2

                lax.fori_loop(0, SW // 16, ib, 0)
                pltpu.sync_copy(emv, acc.at[locv], add=True)
                return carry

            lax.fori_loop(0, EPS_SC // SW, body, 0)
            plsc.subcore_barrier()

            @pl.when((q < 3) | (s < 15))
            def _():
                pltpu.sync_copy(acc.at[pl.ds(s * 392, 392)],
                                out_hbm.at[pl.ds(lo + s * 392, 392)])

            @pl.when((q == 3) & (s == 15))
            def _():
                pltpu.sync_copy(acc.at[pl.ds(15 * 392, 304)],
                                out_hbm.at[pl.ds(lo + 15 * 392, 304)])

            plsc.subcore_barrier()

    return k


def _sc_scatter(em, dsth):
    return _scatter_kernel()(em, dsth)


BNP = 1000
BE = 2000


def _full_spec(shape):
    nd = len(shape)
    return pl.BlockSpec(shape, lambda i: (0,) * nd)


def _proj_body(x_ref, w_ref, b_ref, o_ref):
    o_ref[...] = jnp.dot(x_ref[...], w_ref[...],
                         preferred_element_type=jnp.float32) + b_ref[...]


def _proj(x2, w2, b2):
    return pl.pallas_call(
        _proj_body,
        grid=(NP // BNP,),
        in_specs=[
            pl.BlockSpec((BNP, 256), lambda i: (i, 0)),
            _full_spec((256, HP)),
            _full_spec((1, HP)),
        ],
        out_specs=pl.BlockSpec((BNP, HP), lambda i: (i, 0)),
        out_shape=jax.ShapeDtypeStruct((NP, HP), jnp.float32),
    )(x2, w2, b2)


def _sel(g, p):
    return g[:, 0:H] * (1.0 - p) + g[:, H:2 * H] * p


def _pack_em(em, p):
    return jnp.concatenate([em * (1.0 - p), em * p], axis=1)


def _edge_first_body(gd, gs, r, pd, ps, wgx, bg, rvg, wm1, bm, rvm, w2, b2,
                     em):
    pdv = pd[...]
    xiv = _sel(gd[...], pdv)
    xjv = _sel(gs[...], ps[...])
    X = jnp.concatenate([xiv, xjv], axis=1)
    rr = r[...]
    hg = jnp.tanh(jnp.dot(X, wgx[...], preferred_element_type=jnp.float32)
                  + bg[...] + rr * rvg[...])
    hm = jax.nn.relu(jnp.dot(xjv, wm1[...], preferred_element_type=jnp.float32)
                     + bm[...] + rr * rvm[...])
    S = jnp.dot(jnp.concatenate([hg, hm], axis=1), w2[...],
                preferred_element_type=jnp.float32) + b2[...]
    em[...] = _pack_em(jax.nn.sigmoid(S[:, 0:1]) * S[:, 1:1 + H], pdv)


def _edge_first(gd, gs, r, pd, ps, wgx, bg, rvg, wm1, bm, rvm, w2, b2):
    return pl.pallas_call(
        _edge_first_body,
        grid=(E // BE,),
        in_specs=[
            pl.BlockSpec((BE, HP), lambda i: (i, 0)),
            pl.BlockSpec((BE, HP), lambda i: (i, 0)),
            pl.BlockSpec((BE, 1), lambda i: (i, 0)),
            pl.BlockSpec((BE, 1), lambda i: (i, 0)),
            pl.BlockSpec((BE, 1), lambda i: (i, 0)),
            _full_spec((128, H)), _full_spec((1, H)), _full_spec((1, H)),
            _full_spec((H, H)), _full_spec((1, H)), _full_spec((1, H)),
            _full_spec((128, 1 + H)), _full_spec((1, 1 + H)),
        ],
        out_specs=pl.BlockSpec((BE, HP), lambda i: (i, 0)),
        out_shape=jax.ShapeDtypeStruct((E, HP), jnp.float32),
    )(gd, gs, r, pd, ps, wgx, bg, rvg, wm1, bm, rvm, w2, b2)


def _edge_mid_body(gd, gs, r, pd, ps, wgc, bgc, wc2, bc2, rvg, wm1, bm, rvm,
                   w2, b2, em, rn_ref):
    pdv = pd[...]
    xiv = _sel(gd[...], pdv)
    xjv = _sel(gs[...], ps[...])
    X = jnp.concatenate([xiv, xjv], axis=1)
    P = jnp.dot(X, wgc[...], preferred_element_type=jnp.float32) + bgc[...]
    hc = jnp.tanh(P[:, 0:H])
    t = jnp.tanh(jnp.dot(hc, wc2[...], preferred_element_type=jnp.float32)
                 + bc2[...])
    rn = 0.7 * r[...] + 0.3 * t
    hg = jnp.tanh(P[:, H:2 * H] + rn * rvg[...])
    hm = jax.nn.relu(jnp.dot(xjv, wm1[...], preferred_element_type=jnp.float32)
                     + bm[...] + rn * rvm[...])
    S = jnp.dot(jnp.concatenate([hg, hm], axis=1), w2[...],
                preferred_element_type=jnp.float32) + b2[...]
    em[...] = _pack_em(jax.nn.sigmoid(S[:, 0:1]) * S[:, 1:1 + H], pdv)
    rn_ref[...] = rn


def _edge_mid(gd, gs, r, pd, ps, wgc, bgc, wc2, bc2, rvg, wm1, bm, rvm,
              w2, b2):
    return pl.pallas_call(
        _edge_mid_body,
        grid=(E // BE,),
        in_specs=[
            pl.BlockSpec((BE, HP), lambda i: (i, 0)),
            pl.BlockSpec((BE, HP), lambda i: (i, 0)),
            pl.BlockSpec((BE, 1), lambda i: (i, 0)),
            pl.BlockSpec((BE, 1), lambda i: (i, 0)),
            pl.BlockSpec((BE, 1), lambda i: (i, 0)),
            _full_spec((128, 128)), _full_spec((1, 128)),
            _full_spec((H, 1)), _full_spec((1, 1)),
            _full_spec((1, H)),
            _full_spec((H, H)), _full_spec((1, H)), _full_spec((1, H)),
            _full_spec((128, 1 + H)), _full_spec((1, 1 + H)),
        ],
        out_specs=[
            pl.BlockSpec((BE, HP), lambda i: (i, 0)),
            pl.BlockSpec((BE, 1), lambda i: (i, 0)),
        ],
        out_shape=[
            jax.ShapeDtypeStruct((E, HP), jnp.float32),
            jax.ShapeDtypeStruct((E, 1), jnp.float32),
        ],
    )(gd, gs, r, pd, ps, wgc, bgc, wc2, bc2, rvg, wm1, bm, rvm, w2, b2)


def _edge_final_body(gd, gs, r, pd, ps, wce, bce, wc2, bc2, rve, we2, be2,
                     rn_ref, ew):
    xiv = _sel(gd[...], pd[...])
    xjv = _sel(gs[...], ps[...])
    X = jnp.concatenate([xiv, xjv], axis=1)
    P = jnp.dot(X, wce[...], preferred_element_type=jnp.float32) + bce[...]
    hc = jnp.tanh(P[:, 0:H])
    t = jnp.tanh(jnp.dot(hc, wc2[...], preferred_element_type=jnp.float32)
                 + bc2[...])
    rn = 0.7 * r[...] + 0.3 * t
    hw = jax.nn.relu(P[:, H:2 * H] + rn * rve[...])
    ew[...] = jax.nn.sigmoid(jnp.dot(hw, we2[...],
                                     preferred_element_type=jnp.float32)
                             + be2[...])
    rn_ref[...] = rn


def _edge_final(gd, gs, r, pd, ps, wce, bce, wc2, bc2, rve, we2, be2):
    return pl.pallas_call(
        _edge_final_body,
        grid=(E // BE,),
        in_specs=[
            pl.BlockSpec((BE, HP), lambda i: (i, 0)),
            pl.BlockSpec((BE, HP), lambda i: (i, 0)),
            pl.BlockSpec((BE, 1), lambda i: (i, 0)),
            pl.BlockSpec((BE, 1), lambda i: (i, 0)),
            pl.BlockSpec((BE, 1), lambda i: (i, 0)),
            _full_spec((128, 128)), _full_spec((1, 128)),
            _full_spec((H, 1)), _full_spec((1, 1)),
            _full_spec((1, H)),
            _full_spec((H, 1)), _full_spec((1, 1)),
        ],
        out_specs=[
            pl.BlockSpec((BE, 1), lambda i: (i, 0)),
            pl.BlockSpec((BE, 1), lambda i: (i, 0)),
        ],
        out_shape=[
            jax.ShapeDtypeStruct((E, 1), jnp.float32),
            jax.ShapeDtypeStruct((E, 1), jnp.float32),
        ],
    )(gd, gs, r, pd, ps, wce, bce, wc2, bc2, rve, we2, be2)


def _ln_half(y, sc, bi):
    mu = jnp.mean(y, axis=1, keepdims=True)
    var = jnp.mean((y - mu) ** 2, axis=1, keepdims=True)
    return (y - mu) / jnp.sqrt(var + 1e-5) * sc + bi


def _update_body(h, aggr, wut, wub, bu2, sc, bi, o_ref):
    hh = h[...]
    u = jax.nn.relu(jnp.dot(hh, wut[...], preferred_element_type=jnp.float32)
                    + jnp.dot(aggr[...], wub[...],
                              preferred_element_type=jnp.float32)
                    + bu2[...])
    y = hh + u
    scv = sc[...]
    biv = bi[...]
    o_ref[...] = jnp.concatenate(
        [_ln_half(y[:, 0:H], scv, biv), _ln_half(y[:, H:2 * H], scv, biv)],
        axis=1)


def _update(h, aggr, wut, wub, bu2, sc, bi):
    return pl.pallas_call(
        _update_body,
        grid=(NP // BNP,),
        in_specs=[
            pl.BlockSpec((BNP, HP), lambda i: (i, 0)),
            pl.BlockSpec((BNP, HP), lambda i: (i, 0)),
            _full_spec((HP, HP)), _full_spec((HP, HP)), _full_spec((1, HP)),
            _full_spec((1, H)), _full_spec((1, H)),
        ],
        out_specs=pl.BlockSpec((BNP, HP), lambda i: (i, 0)),
        out_shape=jax.ShapeDtypeStruct((NP, HP), jnp.float32),
    )(h, aggr, wut, wub, bu2, sc, bi)


def _final_body(h, wn1, bn1, wn2, bn2, wg1, bg1, wg2, bg2,
                risk, hsum, gsum):
    pid = pl.program_id(0)
    hh = h[...]
    q = jax.nn.relu(jnp.dot(hh, wn1[...], preferred_element_type=jnp.float32)
                    + bn1[...])
    risk[...] = jax.nn.sigmoid(jnp.dot(q, wn2[...],
                                       preferred_element_type=jnp.float32)
                               + bn2[...])
    blksum = jnp.sum(hh[:, 0:H] + hh[:, H:2 * H], axis=0, keepdims=True)

    @pl.when(pid == 0)
    def _():
        hsum[...] = blksum

    @pl.when(pid > 0)
    def _():
        hsum[...] = hsum[...] + blksum

    @pl.when(pid == NP // BNP - 1)
    def _():
        m = hsum[...] * (1.0 / N)
        g = jax.nn.relu(jnp.dot(m, wg1[...], preferred_element_type=jnp.float32)
                        + bg1[...])
        gsum[...] = jnp.dot(g, wg2[...], preferred_element_type=jnp.float32) \
            + bg2[...]


def _node_final(h, wn1, bn1, wn2, bn2, wg1, bg1, wg2, bg2):
    return pl.pallas_call(
        _final_body,
        grid=(NP // BNP,),
        in_specs=[
            pl.BlockSpec((BNP, HP), lambda i: (i, 0)),
            _full_spec((HP, H)), _full_spec((1, H)),
            _full_spec((H, 2)), _full_spec((1, 2)),
            _full_spec((H, 32)), _full_spec((1, 32)),
            _full_spec((32, 3)), _full_spec((1, 3)),
        ],
        out_specs=[
            pl.BlockSpec((BNP, 2), lambda i: (i, 0)),
            _full_spec((1, H)),
            _full_spec((1, 3)),
        ],
        out_shape=[
            jax.ShapeDtypeStruct((NP, 2), jnp.float32),
            jax.ShapeDtypeStruct((1, H), jnp.float32),
            jax.ShapeDtypeStruct((1, 3), jnp.float32),
        ],
    )(h, wn1, bn1, wn2, bn2, wg1, bg1, wg2, bg2)


def _row(v):
    return v.reshape(1, -1)


def _bd(w):
    a, b = w.shape
    z = jnp.zeros((a, b), jnp.float32)
    return jnp.concatenate([
        jnp.concatenate([w, z], axis=1),
        jnp.concatenate([z, w], axis=1),
    ], axis=0)


def _pair(v):
    return jnp.concatenate([v, v]).reshape(1, -1)


def _pack_gm(lp):
    Wg, bg = lp['gate1']
    wg2, bg2 = lp['gate2']
    Wm, bm = lp['msg1']
    Wm2, bm2 = lp['msg2']
    wgx = Wg[0:2 * H]
    rvg = _row(Wg[2 * H])
    wm1 = Wm[0:H]
    rvm = _row(Wm[H])
    w2 = jnp.concatenate([
        jnp.concatenate([wg2, jnp.zeros((H, H), jnp.float32)], axis=1),
        jnp.concatenate([jnp.zeros((H, 1), jnp.float32), Wm2], axis=1),
    ], axis=0)
    b2 = jnp.concatenate([bg2, bm2]).reshape(1, 1 + H)
    return wgx, _row(bg), rvg, wm1, _row(bm), rvm, w2, b2


def _pack_curv(lp):
    Wc, bc = lp['curv1']
    wc2, bc2 = lp['curv2']
    wcx = jnp.concatenate([Wc[H:2 * H], Wc[0:H]], axis=0)
    return wcx, bc, wc2, bc2.reshape(1, 1)


def kernel(x, edge_index, ricci_curvature, params):
    src = edge_index[0]
    dst = edge_index[1]
    srch = lax.shift_right_logical(src, 1)
    dsth = lax.shift_right_logical(dst, 1)
    psrc = (src & 1).astype(jnp.float32).reshape(E, 1)
    pdst = (dst & 1).astype(jnp.float32).reshape(E, 1)
    r0 = ricci_curvature.reshape(E, 1)
    L = params['layers']

    win, bin_ = params['input_proj']
    x2 = x.reshape(NP, 256)
    h0 = _proj(x2, _bd(win), _pair(bin_))

    gs0, gd0 = _sc_gather(h0, srch, dsth)
    wgx, bg, rvg, wm1, bm, rvm, w2, b2 = _pack_gm(L[0])
    em1 = _edge_first(gd0, gs0, r0, pdst, psrc,
                      wgx, bg, rvg, wm1, bm, rvm, w2, b2)
    a1 = _sc_scatter(em1, dsth)

    def upd(h, a, lp):
        wu, bu = lp['upd']
        return _update(h, a, _bd(wu[0:H]), _bd(wu[H:2 * H]), _pair(bu),
                       _row(lp['ln_scale']), _row(lp['ln_bias']))

    h = upd(h0, a1, L[0])
    r = r0
    for li in (1, 2):
        gs, gd = _sc_gather(h, srch, dsth)
        wcx, bc, wc2, bc2 = _pack_curv(L[li - 1])
        wgx, bg, rvg, wm1, bm, rvm, w2, b2 = _pack_gm(L[li])
        wgc = jnp.concatenate([wcx, wgx], axis=1)
        bgc = jnp.concatenate([bc, bg.reshape(-1)]).reshape(1, 2 * H)
        em, r = _edge_mid(gd, gs, r, pdst, psrc, wgc, bgc, wc2, bc2, rvg,
                          wm1, bm, rvm, w2, b2)
        a = _sc_scatter(em, dsth)
        h = upd(h, a, L[li])

    gs, gd = _sc_gather(h, srch, dsth)
    wcx, bc, wc2, bc2 = _pack_curv(L[2])
    We, be = params['ew1']
    wex = jnp.concatenate([We[H:2 * H], We[0:H]], axis=0)
    rve = _row(We[2 * H])
    we2, be2 = params['ew2']
    wce = jnp.concatenate([wcx, wex], axis=1)
    bce = jnp.concatenate([bc, be]).reshape(1, 2 * H)
    r3, ewp = _edge_final(gd, gs, r, pdst, psrc, wce, bce, wc2, bc2, rve,
                          we2, be2.reshape(1, 1))

    wn1, bn1 = params['nr1']
    wn2, bn2 = params['nr2']
    wg1, bg1 = params['gs1']
    wg2, bg2 = params['gs2']
    wn2p = jnp.concatenate([
        jnp.concatenate([wn2, jnp.zeros((32, 1), jnp.float32)], axis=1),
        jnp.concatenate([jnp.zeros((32, 1), jnp.float32), wn2], axis=1),
    ], axis=0)
    risk, _, gsum = _node_final(
        h, _bd(wn1), _pair(bn1), wn2p,
        jnp.concatenate([bn2, bn2]).reshape(1, 2),
        wg1, _row(bg1), wg2, _row(bg2))

    return (h.reshape(N, H), ewp.reshape(E), risk.reshape(N),
            gsum.reshape(3), r3.reshape(E))


# speedup vs baseline: 1.8391x; 1.1016x over previous
import functools

import jax
import jax.numpy as jnp
from jax import lax
from jax.experimental import pallas as pl
from jax.experimental.pallas import tpu as pltpu
from jax.experimental.pallas import tpu_sc as plsc

N = 50000
E = 800000
H = 64
HP = 128
NP = N // 2

NC = 2
NS = 16
NW = NC * NS

GW = 200
GPW = E // NW


@functools.lru_cache(maxsize=None)
def _gather_kernel():
    mesh = plsc.VectorSubcoreMesh(core_axis_name="c", subcore_axis_name="s")

    @functools.partial(
        pl.kernel,
        mesh=mesh,
        out_type=(
            jax.ShapeDtypeStruct((E, HP), jnp.float32),
            jax.ShapeDtypeStruct((E, HP), jnp.float32),
        ),
        scratch_types=[
            pltpu.VMEM((GPW,), jnp.int32),
            pltpu.VMEM((GW, HP), jnp.float32),
            pltpu.VMEM((GW, HP), jnp.float32),
            pltpu.SemaphoreType.DMA,
            pltpu.SemaphoreType.DMA,
        ],
    )
    def k(table, src_hbm, dst_hbm, out_src, out_dst, idx_v, ra, rb, sa, sb):
        wid = lax.axis_index("s") * NC + lax.axis_index("c")
        base = wid * GPW
        nwin = GPW // GW

        def gat(w, buf, sem):
            return pltpu.async_copy(
                table.at[idx_v.at[pl.ds(w * GW, GW)]], buf, sem)

        def wait(w, buf, sem):
            pltpu.make_async_copy(
                table.at[idx_v.at[pl.ds(w * GW, GW)]], buf, sem).wait()

        for idx_hbm, out in ((src_hbm, out_src), (dst_hbm, out_dst)):
            pltpu.sync_copy(idx_hbm.at[pl.ds(base, GPW)], idx_v)
            gat(0, ra, sa)

            def body(i, carry):
                w0 = 2 * i
                gat(w0 + 1, rb, sb)
                wait(w0, ra, sa)
                pltpu.sync_copy(ra, out.at[pl.ds(base + w0 * GW, GW)])
                gat(w0 + 2, ra, sa)
                wait(w0 + 1, rb, sb)
                pltpu.sync_copy(rb, out.at[pl.ds(base + (w0 + 1) * GW, GW)])
                return carry

            lax.fori_loop(0, (nwin - 1) // 2, body, 0)
            wait(nwin - 1, ra, sa)
            pltpu.sync_copy(ra, out.at[pl.ds(base + (nwin - 1) * GW, GW)])

    return k


def _sc_gather(table, srch, dsth):
    return _gather_kernel()(table, srch, dsth)


CORE0_ROWS = 12544
CORE1_ROWS = NP - CORE0_ROWS
DUMMY_ROWS = 512
SP_ROWS = CORE0_ROWS + DUMMY_ROWS
SW = 80
EPS_SC = E // NS


@functools.lru_cache(maxsize=None)
def _scatter_kernel():
    mesh = plsc.VectorSubcoreMesh(core_axis_name="c", subcore_axis_name="s")

    @functools.partial(
        pl.kernel,
        mesh=mesh,
        out_type=jax.ShapeDtypeStruct((NP, HP), jnp.float32),
        scratch_types=[
            pltpu.VMEM((SW,), jnp.int32),
            pltpu.VMEM((SW,), jnp.int32),
            pltpu.VMEM((SW,), jnp.int32),
            pltpu.VMEM((SW,), jnp.int32),
            pltpu.VMEM((SW, HP), jnp.float32),
            pltpu.VMEM((SW, HP), jnp.float32),
            pltpu.VMEM_SHARED((SP_ROWS, HP), jnp.float32),
            pltpu.SemaphoreType.DMA,
            pltpu.SemaphoreType.DMA,
        ],
    )
    def k(em_hbm, dsth_hbm, out_hbm, da, db, la, lb, ea, eb, acc, sa, sb):
        c = lax.axis_index("c")
        s = lax.axis_index("s")
        lanes = lax.iota(jnp.int32, 16)
        zv = jnp.zeros((16,), jnp.float32)
        nwin = EPS_SC // SW
        lo = c * CORE0_ROWS
        hi = jnp.where(c == 0, CORE0_ROWS, CORE1_ROWS)

        def reads(w, dv, ev):
            off = s * EPS_SC + w * SW
            pltpu.sync_copy(dsth_hbm.at[pl.ds(off, SW)], dv)
            pltpu.sync_copy(em_hbm.at[pl.ds(off, SW)], ev)

        def compute(w, dv, lv):
            def ib(i, carry2):
                d = dv[pl.ds(i * 16, 16)]
                local = d - lo
                inb = (local >= 0) & (local < hi)
                grp = (w * (SW // 16) + i) & (DUMMY_ROWS // 16 - 1)
                dummy = CORE0_ROWS + grp * 16 + lanes
                lv[pl.ds(i * 16, 16)] = jnp.where(inb, local, dummy)
                return carry2

            lax.fori_loop(0, SW // 16, ib, 0)

        def add_start(ev, lv, sem):
            pltpu.async_copy(ev, acc.at[lv], sem, add=True)

        def add_wait(ev, lv, sem):
            pltpu.make_async_copy(ev, acc.at[lv], sem).wait()

        def zrow(i, carry):
            for j in range(HP // 16):
                ea[i, pl.ds(j * 16, 16)] = zv
            return carry

        lax.fori_loop(0, SW, zrow, 0)
        zbase = s * (SP_ROWS // NS)

        def zcp(i, carry):
            pltpu.sync_copy(ea, acc.at[pl.ds(zbase + i * SW, SW)])
            return carry

        lax.fori_loop(0, 10, zcp, 0)
        pltpu.sync_copy(ea.at[pl.ds(0, 16)],
                        acc.at[pl.ds(zbase + 10 * SW, 16)])
        plsc.subcore_barrier()

        reads(0, da, ea)
        compute(0, da, la)
        add_start(ea, la, sa)

        def body(i, carry):
            w1 = 2 * i + 1

            @pl.when(i > 0)
            def _():
                add_wait(eb, lb, sb)

            reads(w1, db, eb)
            compute(w1, db, lb)
            add_start(eb, lb, sb)
            add_wait(ea, la, sa)
            reads(w1 + 1, da, ea)
            compute(w1 + 1, da, la)
            add_start(ea, la, sa)
            return carry

        lax.fori_loop(0, (nwin - 1) // 2, body, 0)
        add_wait(ea, la, sa)
        add_wait(eb, lb, sb)
        plsc.subcore_barrier()

        @pl.when(c == 0)
        def _():
            pltpu.sync_copy(acc.at[pl.ds(s * 784, 784)],
                            out_hbm.at[pl.ds(s * 784, 784)])

        @pl.when((c == 1) & (s < 15))
        def _():
            pltpu.sync_copy(acc.at[pl.ds(s * 776, 776)],
                            out_hbm.at[pl.ds(CORE0_ROWS + s * 776, 776)])

        @pl.when((c == 1) & (s == 15))
        def _():
            pltpu.sync_copy(acc.at[pl.ds(15 * 776, 816)],
                            out_hbm.at[pl.ds(CORE0_ROWS + 15 * 776, 816)])

    return k


def _sc_scatter(em, dsth):
    return _scatter_kernel()(em, dsth)


BNP = 1000
BE = 2000


def _full_spec(shape):
    nd = len(shape)
    return pl.BlockSpec(shape, lambda i: (0,) * nd)


def _proj_body(x_ref, w_ref, b_ref, o_ref):
    o_ref[...] = jnp.dot(x_ref[...], w_ref[...],
                         preferred_element_type=jnp.float32) + b_ref[...]


def _proj(x2, w2, b2):
    return pl.pallas_call(
        _proj_body,
        grid=(NP // BNP,),
        in_specs=[
            pl.BlockSpec((BNP, 256), lambda i: (i, 0)),
            _full_spec((256, HP)),
            _full_spec((1, HP)),
        ],
        out_specs=pl.BlockSpec((BNP, HP), lambda i: (i, 0)),
        out_shape=jax.ShapeDtypeStruct((NP, HP), jnp.float32),
    )(x2, w2, b2)


def _sel(g, p):
    return g[:, 0:H] * (1.0 - p) + g[:, H:2 * H] * p


def _pack_em(em, p):
    return jnp.concatenate([em * (1.0 - p), em * p], axis=1)


def _edge_first_body(gd, gs, r, pd, ps, wgx, bg, rvg, wm1, bm, rvm, w2, b2,
                     em):
    pdv = pd[...]
    xiv = _sel(gd[...], pdv)
    xjv = _sel(gs[...], ps[...])
    X = jnp.concatenate([xiv, xjv], axis=1)
    rr = r[...]
    hg = jnp.tanh(jnp.dot(X, wgx[...], preferred_element_type=jnp.float32)
                  + bg[...] + rr * rvg[...])
    hm = jax.nn.relu(jnp.dot(xjv, wm1[...], preferred_element_type=jnp.float32)
                     + bm[...] + rr * rvm[...])
    S = jnp.dot(jnp.concatenate([hg, hm], axis=1), w2[...],
                preferred_element_type=jnp.float32) + b2[...]
    em[...] = _pack_em(jax.nn.sigmoid(S[:, 0:1]) * S[:, 1:1 + H], pdv)


def _edge_first(gd, gs, r, pd, ps, wgx, bg, rvg, wm1, bm, rvm, w2, b2):
    return pl.pallas_call(
        _edge_first_body,
        grid=(E // BE,),
        in_specs=[
            pl.BlockSpec((BE, HP), lambda i: (i, 0)),
            pl.BlockSpec((BE, HP), lambda i: (i, 0)),
            pl.BlockSpec((BE, 1), lambda i: (i, 0)),
            pl.BlockSpec((BE, 1), lambda i: (i, 0)),
            pl.BlockSpec((BE, 1), lambda i: (i, 0)),
            _full_spec((128, H)), _full_spec((1, H)), _full_spec((1, H)),
            _full_spec((H, H)), _full_spec((1, H)), _full_spec((1, H)),
            _full_spec((128, 1 + H)), _full_spec((1, 1 + H)),
        ],
        out_specs=pl.BlockSpec((BE, HP), lambda i: (i, 0)),
        out_shape=jax.ShapeDtypeStruct((E, HP), jnp.float32),
    )(gd, gs, r, pd, ps, wgx, bg, rvg, wm1, bm, rvm, w2, b2)


def _edge_mid_body(gd, gs, r, pd, ps, wgc, bgc, wc2, bc2, rvg, wm1, bm, rvm,
                   w2, b2, em, rn_ref):
    pdv = pd[...]
    xiv = _sel(gd[...], pdv)
    xjv = _sel(gs[...], ps[...])
    X = jnp.concatenate([xiv, xjv], axis=1)
    P = jnp.dot(X, wgc[...], preferred_element_type=jnp.float32) + bgc[...]
    hc = jnp.tanh(P[:, 0:H])
    t = jnp.tanh(jnp.dot(hc, wc2[...], preferred_element_type=jnp.float32)
                 + bc2[...])
    rn = 0.7 * r[...] + 0.3 * t
    hg = jnp.tanh(P[:, H:2 * H] + rn * rvg[...])
    hm = jax.nn.relu(jnp.dot(xjv, wm1[...], preferred_element_type=jnp.float32)
                     + bm[...] + rn * rvm[...])
    S = jnp.dot(jnp.concatenate([hg, hm], axis=1), w2[...],
                preferred_element_type=jnp.float32) + b2[...]
    em[...] = _pack_em(jax.nn.sigmoid(S[:, 0:1]) * S[:, 1:1 + H], pdv)
    rn_ref[...] = rn


def _edge_mid(gd, gs, r, pd, ps, wgc, bgc, wc2, bc2, rvg, wm1, bm, rvm,
              w2, b2):
    return pl.pallas_call(
        _edge_mid_body,
        grid=(E // BE,),
        in_specs=[
            pl.BlockSpec((BE, HP), lambda i: (i, 0)),
            pl.BlockSpec((BE, HP), lambda i: (i, 0)),
            pl.BlockSpec((BE, 1), lambda i: (i, 0)),
            pl.BlockSpec((BE, 1), lambda i: (i, 0)),
            pl.BlockSpec((BE, 1), lambda i: (i, 0)),
            _full_spec((128, 128)), _full_spec((1, 128)),
            _full_spec((H, 1)), _full_spec((1, 1)),
            _full_spec((1, H)),
            _full_spec((H, H)), _full_spec((1, H)), _full_spec((1, H)),
            _full_spec((128, 1 + H)), _full_spec((1, 1 + H)),
        ],
        out_specs=[
            pl.BlockSpec((BE, HP), lambda i: (i, 0)),
            pl.BlockSpec((BE, 1), lambda i: (i, 0)),
        ],
        out_shape=[
            jax.ShapeDtypeStruct((E, HP), jnp.float32),
            jax.ShapeDtypeStruct((E, 1), jnp.float32),
        ],
    )(gd, gs, r, pd, ps, wgc, bgc, wc2, bc2, rvg, wm1, bm, rvm, w2, b2)


def _edge_final_body(gd, gs, r, pd, ps, wce, bce, wc2, bc2, rve, we2, be2,
                     rn_ref, ew):
    xiv = _sel(gd[...], pd[...])
    xjv = _sel(gs[...], ps[...])
    X = jnp.concatenate([xiv, xjv], axis=1)
    P = jnp.dot(X, wce[...], preferred_element_type=jnp.float32) + bce[...]
    hc = jnp.tanh(P[:, 0:H])
    t = jnp.tanh(jnp.dot(hc, wc2[...], preferred_element_type=jnp.float32)
                 + bc2[...])
    rn = 0.7 * r[...] + 0.3 * t
    hw = jax.nn.relu(P[:, H:2 * H] + rn * rve[...])
    ew[...] = jax.nn.sigmoid(jnp.dot(hw, we2[...],
                                     preferred_element_type=jnp.float32)
                             + be2[...])
    rn_ref[...] = rn


def _edge_final(gd, gs, r, pd, ps, wce, bce, wc2, bc2, rve, we2, be2):
    return pl.pallas_call(
        _edge_final_body,
        grid=(E // BE,),
        in_specs=[
            pl.BlockSpec((BE, HP), lambda i: (i, 0)),
            pl.BlockSpec((BE, HP), lambda i: (i, 0)),
            pl.BlockSpec((BE, 1), lambda i: (i, 0)),
            pl.BlockSpec((BE, 1), lambda i: (i, 0)),
            pl.BlockSpec((BE, 1), lambda i: (i, 0)),
            _full_spec((128, 128)), _full_spec((1, 128)),
            _full_spec((H, 1)), _full_spec((1, 1)),
            _full_spec((1, H)),
            _full_spec((H, 1)), _full_spec((1, 1)),
        ],
        out_specs=[
            pl.BlockSpec((BE, 1), lambda i: (i, 0)),
            pl.BlockSpec((BE, 1), lambda i: (i, 0)),
        ],
        out_shape=[
            jax.ShapeDtypeStruct((E, 1), jnp.float32),
            jax.ShapeDtypeStruct((E, 1), jnp.float32),
        ],
    )(gd, gs, r, pd, ps, wce, bce, wc2, bc2, rve, we2, be2)


def _ln_half(y, sc, bi):
    mu = jnp.mean(y, axis=1, keepdims=True)
    var = jnp.mean((y - mu) ** 2, axis=1, keepdims=True)
    return (y - mu) / jnp.sqrt(var + 1e-5) * sc + bi


def _update_body(h, aggr, wut, wub, bu2, sc, bi, o_ref):
    hh = h[...]
    u = jax.nn.relu(jnp.dot(hh, wut[...], preferred_element_type=jnp.float32)
                    + jnp.dot(aggr[...], wub[...],
                              preferred_element_type=jnp.float32)
                    + bu2[...])
    y = hh + u
    scv = sc[...]
    biv = bi[...]
    o_ref[...] = jnp.concatenate(
        [_ln_half(y[:, 0:H], scv, biv), _ln_half(y[:, H:2 * H], scv, biv)],
        axis=1)


def _update(h, aggr, wut, wub, bu2, sc, bi):
    return pl.pallas_call(
        _update_body,
        grid=(NP // BNP,),
        in_specs=[
            pl.BlockSpec((BNP, HP), lambda i: (i, 0)),
            pl.BlockSpec((BNP, HP), lambda i: (i, 0)),
            _full_spec((HP, HP)), _full_spec((HP, HP)), _full_spec((1, HP)),
            _full_spec((1, H)), _full_spec((1, H)),
        ],
        out_specs=pl.BlockSpec((BNP, HP), lambda i: (i, 0)),
        out_shape=jax.ShapeDtypeStruct((NP, HP), jnp.float32),
    )(h, aggr, wut, wub, bu2, sc, bi)


def _final_body(h, wn1, bn1, wn2, bn2, wg1, bg1, wg2, bg2,
                risk, hsum, gsum):
    pid = pl.program_id(0)
    hh = h[...]
    q = jax.nn.relu(jnp.dot(hh, wn1[...], preferred_element_type=jnp.float32)
                    + bn1[...])
    risk[...] = jax.nn.sigmoid(jnp.dot(q, wn2[...],
                                       preferred_element_type=jnp.float32)
                               + bn2[...])
    blksum = jnp.sum(hh[:, 0:H] + hh[:, H:2 * H], axis=0, keepdims=True)

    @pl.when(pid == 0)
    def _():
        hsum[...] = blksum

    @pl.when(pid > 0)
    def _():
        hsum[...] = hsum[...] + blksum

    @pl.when(pid == NP // BNP - 1)
    def _():
        m = hsum[...] * (1.0 / N)
        g = jax.nn.relu(jnp.dot(m, wg1[...], preferred_element_type=jnp.float32)
                        + bg1[...])
        gsum[...] = jnp.dot(g, wg2[...], preferred_element_type=jnp.float32) \
            + bg2[...]


def _node_final(h, wn1, bn1, wn2, bn2, wg1, bg1, wg2, bg2):
    return pl.pallas_call(
        _final_body,
        grid=(NP // BNP,),
        in_specs=[
            pl.BlockSpec((BNP, HP), lambda i: (i, 0)),
            _full_spec((HP, H)), _full_spec((1, H)),
            _full_spec((H, 2)), _full_spec((1, 2)),
            _full_spec((H, 32)), _full_spec((1, 32)),
            _full_spec((32, 3)), _full_spec((1, 3)),
        ],
        out_specs=[
            pl.BlockSpec((BNP, 2), lambda i: (i, 0)),
            _full_spec((1, H)),
            _full_spec((1, 3)),
        ],
        out_shape=[
            jax.ShapeDtypeStruct((NP, 2), jnp.float32),
            jax.ShapeDtypeStruct((1, H), jnp.float32),
            jax.ShapeDtypeStruct((1, 3), jnp.float32),
        ],
    )(h, wn1, bn1, wn2, bn2, wg1, bg1, wg2, bg2)


def _row(v):
    return v.reshape(1, -1)


def _bd(w):
    a, b = w.shape
    z = jnp.zeros((a, b), jnp.float32)
    return jnp.concatenate([
        jnp.concatenate([w, z], axis=1),
        jnp.concatenate([z, w], axis=1),
    ], axis=0)


def _pair(v):
    return jnp.concatenate([v, v]).reshape(1, -1)


def _pack_gm(lp):
    Wg, bg = lp['gate1']
    wg2, bg2 = lp['gate2']
    Wm, bm = lp['msg1']
    Wm2, bm2 = lp['msg2']
    wgx = Wg[0:2 * H]
    rvg = _row(Wg[2 * H])
    wm1 = Wm[0:H]
    rvm = _row(Wm[H])
    w2 = jnp.concatenate([
        jnp.concatenate([wg2, jnp.zeros((H, H), jnp.float32)], axis=1),
        jnp.concatenate([jnp.zeros((H, 1), jnp.float32), Wm2], axis=1),
    ], axis=0)
    b2 = jnp.concatenate([bg2, bm2]).reshape(1, 1 + H)
    return wgx, _row(bg), rvg, wm1, _row(bm), rvm, w2, b2


def _pack_curv(lp):
    Wc, bc = lp['curv1']
    wc2, bc2 = lp['curv2']
    wcx = jnp.concatenate([Wc[H:2 * H], Wc[0:H]], axis=0)
    return wcx, bc, wc2, bc2.reshape(1, 1)


def kernel(x, edge_index, ricci_curvature, params):
    src = edge_index[0]
    dst = edge_index[1]
    srch = lax.shift_right_logical(src, 1)
    dsth = lax.shift_right_logical(dst, 1)
    psrc = (src & 1).astype(jnp.float32).reshape(E, 1)
    pdst = (dst & 1).astype(jnp.float32).reshape(E, 1)
    r0 = ricci_curvature.reshape(E, 1)
    L = params['layers']

    win, bin_ = params['input_proj']
    x2 = x.reshape(NP, 256)
    h0 = _proj(x2, _bd(win), _pair(bin_))

    gs0, gd0 = _sc_gather(h0, srch, dsth)
    wgx, bg, rvg, wm1, bm, rvm, w2, b2 = _pack_gm(L[0])
    em1 = _edge_first(gd0, gs0, r0, pdst, psrc,
                      wgx, bg, rvg, wm1, bm, rvm, w2, b2)
    a1 = _sc_scatter(em1, dsth)

    def upd(h, a, lp):
        wu, bu = lp['upd']
        return _update(h, a, _bd(wu[0:H]), _bd(wu[H:2 * H]), _pair(bu),
                       _row(lp['ln_scale']), _row(lp['ln_bias']))

    h = upd(h0, a1, L[0])
    r = r0
    for li in (1, 2):
        gs, gd = _sc_gather(h, srch, dsth)
        wcx, bc, wc2, bc2 = _pack_curv(L[li - 1])
        wgx, bg, rvg, wm1, bm, rvm, w2, b2 = _pack_gm(L[li])
        wgc = jnp.concatenate([wcx, wgx], axis=1)
        bgc = jnp.concatenate([bc, bg.reshape(-1)]).reshape(1, 2 * H)
        em, r = _edge_mid(gd, gs, r, pdst, psrc, wgc, bgc, wc2, bc2, rvg,
                          wm1, bm, rvm, w2, b2)
        a = _sc_scatter(em, dsth)
        h = upd(h, a, L[li])

    gs, gd = _sc_gather(h, srch, dsth)
    wcx, bc, wc2, bc2 = _pack_curv(L[2])
    We, be = params['ew1']
    wex = jnp.concatenate([We[H:2 * H], We[0:H]], axis=0)
    rve = _row(We[2 * H])
    we2, be2 = params['ew2']
    wce = jnp.concatenate([wcx, wex], axis=1)
    bce = jnp.concatenate([bc, be]).reshape(1, 2 * H)
    r3, ewp = _edge_final(gd, gs, r, pdst, psrc, wce, bce, wc2, bc2, rve,
                          we2, be2.reshape(1, 1))

    wn1, bn1 = params['nr1']
    wn2, bn2 = params['nr2']
    wg1, bg1 = params['gs1']
    wg2, bg2 = params['gs2']
    wn2p = jnp.concatenate([
        jnp.concatenate([wn2, jnp.zeros((32, 1), jnp.float32)], axis=1),
        jnp.concatenate([jnp.zeros((32, 1), jnp.float32), wn2], axis=1),
    ], axis=0)
    risk, _, gsum = _node_final(
        h, _bd(wn1), _pair(bn1), wn2p,
        jnp.concatenate([bn2, bn2]).reshape(1, 2),
        wg1, _row(bg1), wg2, _row(bg2))

    return (h.reshape(N, H), ewp.reshape(E), risk.reshape(N),
            gsum.reshape(3), r3.reshape(E))


# speedup vs baseline: 1.9670x; 1.0696x over previous
import functools

import jax
import jax.numpy as jnp
from jax import lax
from jax.experimental import pallas as pl
from jax.experimental.pallas import tpu as pltpu
from jax.experimental.pallas import tpu_sc as plsc

N = 50000
E = 800000
H = 64
HP = 128
NP = N // 2

NC = 2
NS = 16
NW = NC * NS

GW = 200
GPW = E // NW


@functools.lru_cache(maxsize=None)
def _gather_kernel():
    mesh = plsc.VectorSubcoreMesh(core_axis_name="c", subcore_axis_name="s")

    @functools.partial(
        pl.kernel,
        mesh=mesh,
        out_type=(
            jax.ShapeDtypeStruct((E, HP), jnp.float32),
            jax.ShapeDtypeStruct((E, HP), jnp.float32),
        ),
        scratch_types=[
            pltpu.VMEM((GPW,), jnp.int32),
            pltpu.VMEM((GW, HP), jnp.float32),
            pltpu.VMEM((GW, HP), jnp.float32),
            pltpu.SemaphoreType.DMA,
            pltpu.SemaphoreType.DMA,
        ],
    )
    def k(table, src_hbm, dst_hbm, out_src, out_dst, idx_v, ra, rb, sa, sb):
        wid = lax.axis_index("s") * NC + lax.axis_index("c")
        base = wid * GPW
        nwin = GPW // GW

        def gat(w, buf, sem):
            return pltpu.async_copy(
                table.at[idx_v.at[pl.ds(w * GW, GW)]], buf, sem)

        def wait(w, buf, sem):
            pltpu.make_async_copy(
                table.at[idx_v.at[pl.ds(w * GW, GW)]], buf, sem).wait()

        for idx_hbm, out in ((src_hbm, out_src), (dst_hbm, out_dst)):
            pltpu.sync_copy(idx_hbm.at[pl.ds(base, GPW)], idx_v)
            gat(0, ra, sa)

            def body(i, carry):
                w0 = 2 * i
                gat(w0 + 1, rb, sb)
                wait(w0, ra, sa)
                pltpu.sync_copy(ra, out.at[pl.ds(base + w0 * GW, GW)])
                gat(w0 + 2, ra, sa)
                wait(w0 + 1, rb, sb)
                pltpu.sync_copy(rb, out.at[pl.ds(base + (w0 + 1) * GW, GW)])
                return carry

            lax.fori_loop(0, (nwin - 1) // 2, body, 0)
            wait(nwin - 1, ra, sa)
            pltpu.sync_copy(ra, out.at[pl.ds(base + (nwin - 1) * GW, GW)])

    return k


def _sc_gather(table, srch, dsth):
    return _gather_kernel()(table, srch, dsth)


CORE0_ROWS = 12544
CORE1_ROWS = NP - CORE0_ROWS
DUMMY_ROWS = 512
SP_ROWS = CORE0_ROWS + DUMMY_ROWS
SW = 80
EPS_SC = E // NS


@functools.lru_cache(maxsize=None)
def _scatter_kernel():
    mesh = plsc.VectorSubcoreMesh(core_axis_name="c", subcore_axis_name="s")

    @functools.partial(
        pl.kernel,
        mesh=mesh,
        out_type=jax.ShapeDtypeStruct((NP, HP), jnp.float32),
        scratch_types=[
            pltpu.VMEM((SW,), jnp.int32),
            pltpu.VMEM((SW,), jnp.int32),
            pltpu.VMEM((SW,), jnp.int32),
            pltpu.VMEM((SW,), jnp.int32),
            pltpu.VMEM((SW, HP), jnp.float32),
            pltpu.VMEM((SW, HP), jnp.float32),
            pltpu.VMEM_SHARED((SP_ROWS, HP), jnp.float32),
            pltpu.SemaphoreType.DMA,
            pltpu.SemaphoreType.DMA,
            pltpu.SemaphoreType.DMA,
            pltpu.SemaphoreType.DMA,
        ],
    )
    def k(em_hbm, dsth_hbm, out_hbm, da, db, la, lb, ea, eb, acc,
          sa, sb, ra, rb):
        c = lax.axis_index("c")
        s = lax.axis_index("s")
        lanes = lax.iota(jnp.int32, 16)
        zv = jnp.zeros((16,), jnp.float32)
        nwin = EPS_SC // SW
        lo = c * CORE0_ROWS
        hi = jnp.where(c == 0, CORE0_ROWS, CORE1_ROWS)

        def reads_start(w, dv, ev, sem):
            off = s * EPS_SC + w * SW
            pltpu.async_copy(dsth_hbm.at[pl.ds(off, SW)], dv, sem)
            pltpu.async_copy(em_hbm.at[pl.ds(off, SW)], ev, sem)

        def reads_wait(w, dv, ev, sem):
            off = s * EPS_SC + w * SW
            pltpu.make_async_copy(dsth_hbm.at[pl.ds(off, SW)], dv, sem).wait()
            pltpu.make_async_copy(em_hbm.at[pl.ds(off, SW)], ev, sem).wait()

        def compute(w, dv, lv):
            def ib(i, carry2):
                d = dv[pl.ds(i * 16, 16)]
                local = d - lo
                inb = (local >= 0) & (local < hi)
                grp = (w * (SW // 16) + i) & (DUMMY_ROWS // 16 - 1)
                dummy = CORE0_ROWS + grp * 16 + lanes
                lv[pl.ds(i * 16, 16)] = jnp.where(inb, local, dummy)
                return carry2

            lax.fori_loop(0, SW // 16, ib, 0)

        def add_start(ev, lv, sem):
            pltpu.async_copy(ev, acc.at[lv], sem, add=True)

        def add_wait(ev, lv, sem):
            pltpu.make_async_copy(ev, acc.at[lv], sem).wait()

        def zrow(i, carry):
            for j in range(HP // 16):
                ea[i, pl.ds(j * 16, 16)] = zv
            return carry

        lax.fori_loop(0, SW, zrow, 0)
        zbase = s * (SP_ROWS // NS)

        def zcp(i, carry):
            pltpu.sync_copy(ea, acc.at[pl.ds(zbase + i * SW, SW)])
            return carry

        lax.fori_loop(0, 10, zcp, 0)
        pltpu.sync_copy(ea.at[pl.ds(0, 16)],
                        acc.at[pl.ds(zbase + 10 * SW, 16)])
        plsc.subcore_barrier()

        reads_start(0, da, ea, ra)
        reads_wait(0, da, ea, ra)
        compute(0, da, la)
        reads_start(1, db, eb, rb)
        add_start(ea, la, sa)

        def body(i, carry):
            w1 = 2 * i + 1
            reads_wait(w1, db, eb, rb)
            compute(w1, db, lb)
            add_wait(ea, la, sa)
            reads_start(w1 + 1, da, ea, ra)
            add_start(eb, lb, sb)
            reads_wait(w1 + 1, da, ea, ra)
            compute(w1 + 1, da, la)
            add_wait(eb, lb, sb)

            @pl.when(w1 + 2 < nwin)
            def _():
                reads_start(w1 + 2, db, eb, rb)

            add_start(ea, la, sa)
            return carry

        lax.fori_loop(0, (nwin - 1) // 2, body, 0)
        add_wait(ea, la, sa)
        plsc.subcore_barrier()

        @pl.when(c == 0)
        def _():
            pltpu.sync_copy(acc.at[pl.ds(s * 784, 784)],
                            out_hbm.at[pl.ds(s * 784, 784)])

        @pl.when((c == 1) & (s < 15))
        def _():
            pltpu.sync_copy(acc.at[pl.ds(s * 776, 776)],
                            out_hbm.at[pl.ds(CORE0_ROWS + s * 776, 776)])

        @pl.when((c == 1) & (s == 15))
        def _():
            pltpu.sync_copy(acc.at[pl.ds(15 * 776, 816)],
                            out_hbm.at[pl.ds(CORE0_ROWS + 15 * 776, 816)])

    return k


def _sc_scatter(em, dsth):
    return _scatter_kernel()(em, dsth)


BNP = 1000
BE = 2000


def _full_spec(shape):
    nd = len(shape)
    return pl.BlockSpec(shape, lambda i: (0,) * nd)


def _proj_body(x_ref, w_ref, b_ref, o_ref):
    o_ref[...] = jnp.dot(x_ref[...], w_ref[...],
                         preferred_element_type=jnp.float32) + b_ref[...]


def _proj(x2, w2, b2):
    return pl.pallas_call(
        _proj_body,
        grid=(NP // BNP,),
        in_specs=[
            pl.BlockSpec((BNP, 256), lambda i: (i, 0)),
            _full_spec((256, HP)),
            _full_spec((1, HP)),
        ],
        out_specs=pl.BlockSpec((BNP, HP), lambda i: (i, 0)),
        out_shape=jax.ShapeDtypeStruct((NP, HP), jnp.float32),
    )(x2, w2, b2)


def _sel(g, p):
    return g[:, 0:H] * (1.0 - p) + g[:, H:2 * H] * p


def _pack_em(em, p):
    return jnp.concatenate([em * (1.0 - p), em * p], axis=1)


def _edge_first_body(gd, gs, r, pd, ps, wgx, bg, rvg, wm1, bm, rvm, w2, b2,
                     em):
    pdv = pd[...]
    xiv = _sel(gd[...], pdv)
    xjv = _sel(gs[...], ps[...])
    X = jnp.concatenate([xiv, xjv], axis=1)
    rr = r[...]
    hg = jnp.tanh(jnp.dot(X, wgx[...], preferred_element_type=jnp.float32)
                  + bg[...] + rr * rvg[...])
    hm = jax.nn.relu(jnp.dot(xjv, wm1[...], preferred_element_type=jnp.float32)
                     + bm[...] + rr * rvm[...])
    S = jnp.dot(jnp.concatenate([hg, hm], axis=1), w2[...],
                preferred_element_type=jnp.float32) + b2[...]
    em[...] = _pack_em(jax.nn.sigmoid(S[:, 0:1]) * S[:, 1:1 + H], pdv)


def _edge_first(gd, gs, r, pd, ps, wgx, bg, rvg, wm1, bm, rvm, w2, b2):
    return pl.pallas_call(
        _edge_first_body,
        grid=(E // BE,),
        in_specs=[
            pl.BlockSpec((BE, HP), lambda i: (i, 0)),
            pl.BlockSpec((BE, HP), lambda i: (i, 0)),
            pl.BlockSpec((BE, 1), lambda i: (i, 0)),
            pl.BlockSpec((BE, 1), lambda i: (i, 0)),
            pl.BlockSpec((BE, 1), lambda i: (i, 0)),
            _full_spec((128, H)), _full_spec((1, H)), _full_spec((1, H)),
            _full_spec((H, H)), _full_spec((1, H)), _full_spec((1, H)),
            _full_spec((128, 1 + H)), _full_spec((1, 1 + H)),
        ],
        out_specs=pl.BlockSpec((BE, HP), lambda i: (i, 0)),
        out_shape=jax.ShapeDtypeStruct((E, HP), jnp.float32),
    )(gd, gs, r, pd, ps, wgx, bg, rvg, wm1, bm, rvm, w2, b2)


def _edge_mid_body(gd, gs, r, pd, ps, wgc, bgc, wc2, bc2, rvg, wm1, bm, rvm,
                   w2, b2, em, rn_ref):
    pdv = pd[...]
    xiv = _sel(gd[...], pdv)
    xjv = _sel(gs[...], ps[...])
    X = jnp.concatenate([xiv, xjv], axis=1)
    P = jnp.dot(X, wgc[...], preferred_element_type=jnp.float32) + bgc[...]
    hc = jnp.tanh(P[:, 0:H])
    t = jnp.tanh(jnp.dot(hc, wc2[...], preferred_element_type=jnp.float32)
                 + bc2[...])
    rn = 0.7 * r[...] + 0.3 * t
    hg = jnp.tanh(P[:, H:2 * H] + rn * rvg[...])
    hm = jax.nn.relu(jnp.dot(xjv, wm1[...], preferred_element_type=jnp.float32)
                     + bm[...] + rn * rvm[...])
    S = jnp.dot(jnp.concatenate([hg, hm], axis=1), w2[...],
                preferred_element_type=jnp.float32) + b2[...]
    em[...] = _pack_em(jax.nn.sigmoid(S[:, 0:1]) * S[:, 1:1 + H], pdv)
    rn_ref[...] = rn


def _edge_mid(gd, gs, r, pd, ps, wgc, bgc, wc2, bc2, rvg, wm1, bm, rvm,
              w2, b2):
    return pl.pallas_call(
        _edge_mid_body,
        grid=(E // BE,),
        in_specs=[
            pl.BlockSpec((BE, HP), lambda i: (i, 0)),
            pl.BlockSpec((BE, HP), lambda i: (i, 0)),
            pl.BlockSpec((BE, 1), lambda i: (i, 0)),
            pl.BlockSpec((BE, 1), lambda i: (i, 0)),
            pl.BlockSpec((BE, 1), lambda i: (i, 0)),
            _full_spec((128, 128)), _full_spec((1, 128)),
            _full_spec((H, 1)), _full_spec((1, 1)),
            _full_spec((1, H)),
            _full_spec((H, H)), _full_spec((1, H)), _full_spec((1, H)),
            _full_spec((128, 1 + H)), _full_spec((1, 1 + H)),
        ],
        out_specs=[
            pl.BlockSpec((BE, HP), lambda i: (i, 0)),
            pl.BlockSpec((BE, 1), lambda i: (i, 0)),
        ],
        out_shape=[
            jax.ShapeDtypeStruct((E, HP), jnp.float32),
            jax.ShapeDtypeStruct((E, 1), jnp.float32),
        ],
    )(gd, gs, r, pd, ps, wgc, bgc, wc2, bc2, rvg, wm1, bm, rvm, w2, b2)


def _edge_final_body(gd, gs, r, pd, ps, wce, bce, wc2, bc2, rve, we2, be2,
                     rn_ref, ew):
    xiv = _sel(gd[...], pd[...])
    xjv = _sel(gs[...], ps[...])
    X = jnp.concatenate([xiv, xjv], axis=1)
    P = jnp.dot(X, wce[...], preferred_element_type=jnp.float32) + bce[...]
    hc = jnp.tanh(P[:, 0:H])
    t = jnp.tanh(jnp.dot(hc, wc2[...], preferred_element_type=jnp.float32)
                 + bc2[...])
    rn = 0.7 * r[...] + 0.3 * t
    hw = jax.nn.relu(P[:, H:2 * H] + rn * rve[...])
    ew[...] = jax.nn.sigmoid(jnp.dot(hw, we2[...],
                                     preferred_element_type=jnp.float32)
                             + be2[...])
    rn_ref[...] = rn


def _edge_final(gd, gs, r, pd, ps, wce, bce, wc2, bc2, rve, we2, be2):
    return pl.pallas_call(
        _edge_final_body,
        grid=(E // BE,),
        in_specs=[
            pl.BlockSpec((BE, HP), lambda i: (i, 0)),
            pl.BlockSpec((BE, HP), lambda i: (i, 0)),
            pl.BlockSpec((BE, 1), lambda i: (i, 0)),
            pl.BlockSpec((BE, 1), lambda i: (i, 0)),
            pl.BlockSpec((BE, 1), lambda i: (i, 0)),
            _full_spec((128, 128)), _full_spec((1, 128)),
            _full_spec((H, 1)), _full_spec((1, 1)),
            _full_spec((1, H)),
            _full_spec((H, 1)), _full_spec((1, 1)),
        ],
        out_specs=[
            pl.BlockSpec((BE, 1), lambda i: (i, 0)),
            pl.BlockSpec((BE, 1), lambda i: (i, 0)),
        ],
        out_shape=[
            jax.ShapeDtypeStruct((E, 1), jnp.float32),
            jax.ShapeDtypeStruct((E, 1), jnp.float32),
        ],
    )(gd, gs, r, pd, ps, wce, bce, wc2, bc2, rve, we2, be2)


def _ln_half(y, sc, bi):
    mu = jnp.mean(y, axis=1, keepdims=True)
    var = jnp.mean((y - mu) ** 2, axis=1, keepdims=True)
    return (y - mu) / jnp.sqrt(var + 1e-5) * sc + bi


def _update_body(h, aggr, wut, wub, bu2, sc, bi, o_ref):
    hh = h[...]
    u = jax.nn.relu(jnp.dot(hh, wut[...], preferred_element_type=jnp.float32)
                    + jnp.dot(aggr[...], wub[...],
                              preferred_element_type=jnp.float32)
                    + bu2[...])
    y = hh + u
    scv = sc[...]
    biv = bi[...]
    o_ref[...] = jnp.concatenate(
        [_ln_half(y[:, 0:H], scv, biv), _ln_half(y[:, H:2 * H], scv, biv)],
        axis=1)


def _update(h, aggr, wut, wub, bu2, sc, bi):
    return pl.pallas_call(
        _update_body,
        grid=(NP // BNP,),
        in_specs=[
            pl.BlockSpec((BNP, HP), lambda i: (i, 0)),
            pl.BlockSpec((BNP, HP), lambda i: (i, 0)),
            _full_spec((HP, HP)), _full_spec((HP, HP)), _full_spec((1, HP)),
            _full_spec((1, H)), _full_spec((1, H)),
        ],
        out_specs=pl.BlockSpec((BNP, HP), lambda i: (i, 0)),
        out_shape=jax.ShapeDtypeStruct((NP, HP), jnp.float32),
    )(h, aggr, wut, wub, bu2, sc, bi)


def _final_body(h, wn1, bn1, wn2, bn2, wg1, bg1, wg2, bg2,
                risk, hsum, gsum):
    pid = pl.program_id(0)
    hh = h[...]
    q = jax.nn.relu(jnp.dot(hh, wn1[...], preferred_element_type=jnp.float32)
                    + bn1[...])
    risk[...] = jax.nn.sigmoid(jnp.dot(q, wn2[...],
                                       preferred_element_type=jnp.float32)
                               + bn2[...])
    blksum = jnp.sum(hh[:, 0:H] + hh[:, H:2 * H], axis=0, keepdims=True)

    @pl.when(pid == 0)
    def _():
        hsum[...] = blksum

    @pl.when(pid > 0)
    def _():
        hsum[...] = hsum[...] + blksum

    @pl.when(pid == NP // BNP - 1)
    def _():
        m = hsum[...] * (1.0 / N)
        g = jax.nn.relu(jnp.dot(m, wg1[...], preferred_element_type=jnp.float32)
                        + bg1[...])
        gsum[...] = jnp.dot(g, wg2[...], preferred_element_type=jnp.float32) \
            + bg2[...]


def _node_final(h, wn1, bn1, wn2, bn2, wg1, bg1, wg2, bg2):
    return pl.pallas_call(
        _final_body,
        grid=(NP // BNP,),
        in_specs=[
            pl.BlockSpec((BNP, HP), lambda i: (i, 0)),
            _full_spec((HP, H)), _full_spec((1, H)),
            _full_spec((H, 2)), _full_spec((1, 2)),
            _full_spec((H, 32)), _full_spec((1, 32)),
            _full_spec((32, 3)), _full_spec((1, 3)),
        ],
        out_specs=[
            pl.BlockSpec((BNP, 2), lambda i: (i, 0)),
            _full_spec((1, H)),
            _full_spec((1, 3)),
        ],
        out_shape=[
            jax.ShapeDtypeStruct((NP, 2), jnp.float32),
            jax.ShapeDtypeStruct((1, H), jnp.float32),
            jax.ShapeDtypeStruct((1, 3), jnp.float32),
        ],
    )(h, wn1, bn1, wn2, bn2, wg1, bg1, wg2, bg2)


def _row(v):
    return v.reshape(1, -1)


def _bd(w):
    a, b = w.shape
    z = jnp.zeros((a, b), jnp.float32)
    return jnp.concatenate([
        jnp.concatenate([w, z], axis=1),
        jnp.concatenate([z, w], axis=1),
    ], axis=0)


def _pair(v):
    return jnp.concatenate([v, v]).reshape(1, -1)


def _pack_gm(lp):
    Wg, bg = lp['gate1']
    wg2, bg2 = lp['gate2']
    Wm, bm = lp['msg1']
    Wm2, bm2 = lp['msg2']
    wgx = Wg[0:2 * H]
    rvg = _row(Wg[2 * H])
    wm1 = Wm[0:H]
    rvm = _row(Wm[H])
    w2 = jnp.concatenate([
        jnp.concatenate([wg2, jnp.zeros((H, H), jnp.float32)], axis=1),
        jnp.concatenate([jnp.zeros((H, 1), jnp.float32), Wm2], axis=1),
    ], axis=0)
    b2 = jnp.concatenate([bg2, bm2]).reshape(1, 1 + H)
    return wgx, _row(bg), rvg, wm1, _row(bm), rvm, w2, b2


def _pack_curv(lp):
    Wc, bc = lp['curv1']
    wc2, bc2 = lp['curv2']
    wcx = jnp.concatenate([Wc[H:2 * H], Wc[0:H]], axis=0)
    return wcx, bc, wc2, bc2.reshape(1, 1)


def kernel(x, edge_index, ricci_curvature, params):
    src = edge_index[0]
    dst = edge_index[1]
    srch = lax.shift_right_logical(src, 1)
    dsth = lax.shift_right_logical(dst, 1)
    psrc = (src & 1).astype(jnp.float32).reshape(E, 1)
    pdst = (dst & 1).astype(jnp.float32).reshape(E, 1)
    r0 = ricci_curvature.reshape(E, 1)
    L = params['layers']

    win, bin_ = params['input_proj']
    x2 = x.reshape(NP, 256)
    h0 = _proj(x2, _bd(win), _pair(bin_))

    gs0, gd0 = _sc_gather(h0, srch, dsth)
    wgx, bg, rvg, wm1, bm, rvm, w2, b2 = _pack_gm(L[0])
    em1 = _edge_first(gd0, gs0, r0, pdst, psrc,
                      wgx, bg, rvg, wm1, bm, rvm, w2, b2)
    a1 = _sc_scatter(em1, dsth)

    def upd(h, a, lp):
        wu, bu = lp['upd']
        return _update(h, a, _bd(wu[0:H]), _bd(wu[H:2 * H]), _pair(bu),
                       _row(lp['ln_scale']), _row(lp['ln_bias']))

    h = upd(h0, a1, L[0])
    r = r0
    for li in (1, 2):
        gs, gd = _sc_gather(h, srch, dsth)
        wcx, bc, wc2, bc2 = _pack_curv(L[li - 1])
        wgx, bg, rvg, wm1, bm, rvm, w2, b2 = _pack_gm(L[li])
        wgc = jnp.concatenate([wcx, wgx], axis=1)
        bgc = jnp.concatenate([bc, bg.reshape(-1)]).reshape(1, 2 * H)
        em, r = _edge_mid(gd, gs, r, pdst, psrc, wgc, bgc, wc2, bc2, rvg,
                          wm1, bm, rvm, w2, b2)
        a = _sc_scatter(em, dsth)
        h = upd(h, a, L[li])

    gs, gd = _sc_gather(h, srch, dsth)
    wcx, bc, wc2, bc2 = _pack_curv(L[2])
    We, be = params['ew1']
    wex = jnp.concatenate([We[H:2 * H], We[0:H]], axis=0)
    rve = _row(We[2 * H])
    we2, be2 = params['ew2']
    wce = jnp.concatenate([wcx, wex], axis=1)
    bce = jnp.concatenate([bc, be]).reshape(1, 2 * H)
    r3, ewp = _edge_final(gd, gs, r, pdst, psrc, wce, bce, wc2, bc2, rve,
                          we2, be2.reshape(1, 1))

    wn1, bn1 = params['nr1']
    wn2, bn2 = params['nr2']
    wg1, bg1 = params['gs1']
    wg2, bg2 = params['gs2']
    wn2p = jnp.concatenate([
        jnp.concatenate([wn2, jnp.zeros((32, 1), jnp.float32)], axis=1),
        jnp.concatenate([jnp.zeros((32, 1), jnp.float32), wn2], axis=1),
    ], axis=0)
    risk, _, gsum = _node_final(
        h, _bd(wn1), _pair(bn1), wn2p,
        jnp.concatenate([bn2, bn2]).reshape(1, 2),
        wg1, _row(bg1), wg2, _row(bg2))

    return (h.reshape(N, H), ewp.reshape(E), risk.reshape(N),
            gsum.reshape(3), r3.reshape(E))
